# baseline (speedup 1.0000x reference)
import jax
import jax.numpy as jnp
from jax.experimental import pallas as pl
from jax.experimental.pallas import tpu as pltpu

_CH = 8
_R = 65536 // _CH


def _body(fe, a, c, o, cbuf, abuf, cabuf, *sems):
    fe_sems = sems[0:_CH]
    c_sems = sems[_CH:2 * _CH]
    a_sems = sems[2 * _CH:3 * _CH]
    o_sems = sems[3 * _CH:4 * _CH]

    def rows(k):
        return pl.ds(k * _R, _R)

    fe_copies = [
        pltpu.make_async_copy(fe.at[rows(k), :], o.at[rows(k), 0:128], fe_sems[k])
        for k in range(_CH)
    ]
    c_copies = [
        pltpu.make_async_copy(c.at[rows(k), :], cbuf.at[k % 2], c_sems[k])
        for k in range(_CH)
    ]
    a_copies = [
        pltpu.make_async_copy(a.at[rows(k), :], abuf.at[k % 2], a_sems[k])
        for k in range(_CH)
    ]
    o_copies = [
        pltpu.make_async_copy(cabuf.at[k % 2], o.at[rows(k), 128:256], o_sems[k])
        for k in range(_CH)
    ]

    for cp in fe_copies:
        cp.start()
    c_copies[0].start(); a_copies[0].start()
    c_copies[1].start(); a_copies[1].start()
    for k in range(_CH):
        b = k % 2
        c_copies[k].wait()
        a_copies[k].wait()
        if k >= 2:
            o_copies[k - 2].wait()
        cabuf[b, :, 0:64] = cbuf[b, :, :]
        cabuf[b, :, 64:128] = abuf[b, :, :]
        o_copies[k].start()
        if k + 2 < _CH:
            c_copies[k + 2].start()
            a_copies[k + 2].start()
    o_copies[_CH - 2].wait()
    o_copies[_CH - 1].wait()
    for cp in fe_copies:
        cp.wait()


def kernel(decoder_fe_output, decoder_alpha_output, decoder_carbon_output, idx_fe, idx_carbon, idx_alpha, out_dim):
    bsz = decoder_fe_output.shape[0]
    d_fe = decoder_fe_output.shape[1]
    d_a = decoder_alpha_output.shape[1]
    d_c = decoder_carbon_output.shape[1]
    d_out = d_fe + d_a + d_c

    return pl.pallas_call(
        _body,
        in_specs=[
            pl.BlockSpec(memory_space=pl.ANY),
            pl.BlockSpec(memory_space=pl.ANY),
            pl.BlockSpec(memory_space=pl.ANY),
        ],
        out_specs=pl.BlockSpec(memory_space=pl.ANY),
        out_shape=jax.ShapeDtypeStruct((bsz, d_out), decoder_fe_output.dtype),
        scratch_shapes=(
            [pltpu.VMEM((2, _R, 64), jnp.float32)] * 2
            + [pltpu.VMEM((2, _R, 128), jnp.float32)]
            + [pltpu.SemaphoreType.DMA] * (4 * _CH)
        ),
    )(decoder_fe_output, decoder_alpha_output, decoder_carbon_output)


# speedup vs baseline: 10.5343x; 10.5343x over previous
import jax
import jax.numpy as jnp
from jax.experimental import pallas as pl
from jax.experimental.pallas import tpu as pltpu

_CH = 16
_R = 65536 // _CH
_NB = 4


def _body(fe, a, c, o, febuf, cbuf, abuf, obuf, *sems):
    fe_s = sems[0:_NB]
    c_s = sems[_NB:2 * _NB]
    a_s = sems[2 * _NB:3 * _NB]
    o_s = sems[3 * _NB:4 * _NB]

    def rows(k):
        return pl.ds(k * _R, _R)

    def in_copies(k):
        b = k % _NB
        return (
            pltpu.make_async_copy(fe.at[rows(k), :], febuf.at[b], fe_s[b]),
            pltpu.make_async_copy(c.at[rows(k), :], cbuf.at[b], c_s[b]),
            pltpu.make_async_copy(a.at[rows(k), :], abuf.at[b], a_s[b]),
        )

    def out_copy(k):
        b = k % _NB
        return pltpu.make_async_copy(obuf.at[b], o.at[rows(k), :], o_s[b])

    for k in range(_NB):
        for cp in in_copies(k):
            cp.start()
    for k in range(_CH):
        b = k % _NB
        for cp in in_copies(k):
            cp.wait()
        if k >= _NB:
            out_copy(k - _NB).wait()
        obuf[b, :, 0:128] = febuf[b]
        obuf[b, :, 128:192] = cbuf[b]
        obuf[b, :, 192:256] = abuf[b]
        out_copy(k).start()
        if k + _NB < _CH:
            for cp in in_copies(k + _NB):
                cp.start()
    for k in range(_CH - _NB, _CH):
        out_copy(k).wait()


def kernel(decoder_fe_output, decoder_alpha_output, decoder_carbon_output, idx_fe, idx_carbon, idx_alpha, out_dim):
    bsz = decoder_fe_output.shape[0]
    d_fe = decoder_fe_output.shape[1]
    d_a = decoder_alpha_output.shape[1]
    d_c = decoder_carbon_output.shape[1]
    d_out = d_fe + d_a + d_c

    return pl.pallas_call(
        _body,
        in_specs=[
            pl.BlockSpec(memory_space=pl.ANY),
            pl.BlockSpec(memory_space=pl.ANY),
            pl.BlockSpec(memory_space=pl.ANY),
        ],
        out_specs=pl.BlockSpec(memory_space=pl.ANY),
        out_shape=jax.ShapeDtypeStruct((bsz, d_out), decoder_fe_output.dtype),
        scratch_shapes=(
            [
                pltpu.VMEM((_NB, _R, d_fe), jnp.float32),
                pltpu.VMEM((_NB, _R, d_c), jnp.float32),
                pltpu.VMEM((_NB, _R, d_a), jnp.float32),
                pltpu.VMEM((_NB, _R, d_out), jnp.float32),
            ]
            + [pltpu.SemaphoreType.DMA] * (4 * _NB)
        ),
    )(decoder_fe_output, decoder_alpha_output, decoder_carbon_output)
